# baseline (speedup 1.0000x reference)
import jax
import jax.numpy as jnp
from jax import lax
from jax.experimental import pallas as pl
from jax.experimental.pallas import tpu as pltpu
from jax.experimental.pallas import tpu_sc as plsc

NS = 16
EB = 512
SB = 2 * EB
GCH = 128
NGC = EB // GCH
HALF = 32
ZR = 200

_NSESS = 25000
_N = 50000
_NSESS_PAD = 25600
_N_PAD = 51200
_EA_PAD = 802816
_ES_PAD = 507904


def _spmm_sc(rows2d, cols2d, vals, x_pair, ndst_pad, nblk):
    ept = nblk * SB
    rpt = ndst_pad // NS
    ncp = rpt // ZR
    mesh = plsc.VectorSubcoreMesh(core_axis_name="c", subcore_axis_name="s")

    NQ = SB // GCH
    NSLOT = 4

    def body(rows_hbm, cols_hbm, vals_hbm, x_hbm, out_hbm,
             acc, gbuf, zbuf, rowv, colv, valv, gsem, ssem, zsem):
        c = lax.axis_index("c")
        s = lax.axis_index("s")

        def _zrow(r, carry):
            zbuf[r, 0:16] = jnp.zeros((16,), jnp.float32)
            zbuf[r, 16:32] = jnp.zeros((16,), jnp.float32)
            return carry
        lax.fori_loop(0, ZR, _zrow, 0)
        zdescs = [
            pltpu.async_copy(
                zbuf, acc.at[pl.ds(pl.multiple_of(s * rpt + k * ZR, ZR), ZR)],
                zsem)
            for k in range(ncp)
        ]
        for d in zdescs:
            d.wait()
        plsc.subcore_barrier()

        xh = x_hbm.at[c]
        dnums = lax.GatherDimensionNumbers(
            offset_dims=(), collapsed_slice_dims=(0,), start_index_map=(0,))

        def _sblk(b, carry):
            e0 = pl.multiple_of(s * ept + b * SB, SB)
            g0 = pl.multiple_of((s * ept) // GCH + b * NQ, NQ)
            pltpu.sync_copy(rows_hbm.at[pl.ds(g0, NQ)], rowv)
            pltpu.sync_copy(cols_hbm.at[pl.ds(g0, NQ)], colv)
            pltpu.sync_copy(vals_hbm.at[pl.ds(e0, SB)], valv)

            def _gather(q):
                return pltpu.async_copy(
                    xh.at[colv.at[q]],
                    gbuf.at[pl.ds((q % NSLOT) * GCH, GCH)], gsem)

            gd = {0: _gather(0)}
            sd = []
            for q in range(NQ):
                gd.pop(q).wait()
                if q + 1 < NQ:
                    if q + 1 >= NSLOT:
                        sd.pop(0).wait()
                    gd[q + 1] = _gather(q + 1)

                slot = (q % NSLOT) * GCH
                vbase = q * GCH

                def _grp(g, carry2, _slot=slot, _vbase=vbase):
                    vv = valv[pl.ds(_vbase + g * 16, 16)]
                    base = _slot + g * 16
                    for k in range(16):
                        sp = lax.gather(
                            vv, jnp.full((16, 1), k, jnp.int32), dnums, (1,),
                            mode=lax.GatherScatterMode.PROMISE_IN_BOUNDS)
                        e = base + k
                        gbuf[e, 0:16] = gbuf[e, 0:16] * sp
                        gbuf[e, 16:32] = gbuf[e, 16:32] * sp
                    return carry2
                lax.fori_loop(0, GCH // 16, _grp, 0)

                sd.append(pltpu.async_copy(gbuf.at[pl.ds(slot, GCH)],
                                           acc.at[rowv.at[q]], ssem,
                                           add=True))
            for d in sd:
                d.wait()
            return carry
        lax.fori_loop(0, nblk, _sblk, 0)
        plsc.subcore_barrier()

        oh = out_hbm.at[c]
        for k in range(ncp):
            off = pl.multiple_of(s * rpt + k * ZR, ZR)
            pltpu.sync_copy(acc.at[pl.ds(off, ZR)], zbuf)
            pltpu.sync_copy(zbuf, oh.at[pl.ds(off, ZR)])

    f = pl.kernel(
        body,
        out_type=jax.ShapeDtypeStruct((2, ndst_pad, HALF), jnp.float32),
        mesh=mesh,
        scratch_types=[
            pltpu.VMEM_SHARED((ndst_pad, HALF), jnp.float32),
            pltpu.VMEM((EB, HALF), jnp.float32),
            pltpu.VMEM((ZR, HALF), jnp.float32),
            pltpu.VMEM((SB // GCH, GCH), jnp.int32),
            pltpu.VMEM((SB // GCH, GCH), jnp.int32),
            pltpu.VMEM((SB,), jnp.float32),
            pltpu.SemaphoreType.DMA,
            pltpu.SemaphoreType.DMA,
            pltpu.SemaphoreType.DMA,
        ],
        compiler_params=pltpu.CompilerParams(use_tc_tiling_on_sc=False),
    )
    return f(rows2d, cols2d, vals, x_pair)


def _sc_gather(s3, bidx, iidx):
    mesh = plsc.VectorSubcoreMesh(core_axis_name="c", subcore_axis_name="s")

    def body(s_hbm, b_hbm, i_hbm, gb_hbm, gi_hbm,
             bi_v, ii_a, ii_b, bbuf, ibuf, sem):
        c = lax.axis_index("c")
        s = lax.axis_index("s")
        src = s_hbm.at[c]
        o64 = pl.multiple_of(s * 64, 64)
        o256 = pl.multiple_of(s * 256, 256)
        pltpu.sync_copy(b_hbm.at[pl.ds(o64, 64)], bi_v)
        pltpu.async_copy(src.at[bi_v], bbuf, sem).wait()
        pltpu.sync_copy(bbuf, gb_hbm.at[c].at[pl.ds(o64, 64)])
        pltpu.sync_copy(i_hbm.at[pl.ds(o256, 128)], ii_a)
        pltpu.sync_copy(i_hbm.at[pl.ds(o256 + 128, 128)], ii_b)
        d0 = pltpu.async_copy(src.at[ii_a], ibuf.at[pl.ds(0, 128)], sem)
        d1 = pltpu.async_copy(src.at[ii_b], ibuf.at[pl.ds(128, 128)], sem)
        d0.wait()
        d1.wait()
        pltpu.sync_copy(ibuf, gi_hbm.at[c].at[pl.ds(o256, 256)])

    f = pl.kernel(
        body,
        out_type=(jax.ShapeDtypeStruct((2, 1024, HALF), jnp.float32),
                  jax.ShapeDtypeStruct((2, 4096, HALF), jnp.float32)),
        mesh=mesh,
        scratch_types=[
            pltpu.VMEM((64,), jnp.int32),
            pltpu.VMEM((128,), jnp.int32),
            pltpu.VMEM((128,), jnp.int32),
            pltpu.VMEM((64, HALF), jnp.float32),
            pltpu.VMEM((256, HALF), jnp.float32),
            pltpu.SemaphoreType.DMA,
        ],
        compiler_params=pltpu.CompilerParams(use_tc_tiling_on_sc=False),
    )
    return f(s3, bidx, iidx)


def _tc_mid(s_pair, W, b):
    R = s_pair.shape[1]
    BR = 2048

    def body(s_ref, w_ref, b_ref, o_ref):
        sblk = jnp.concatenate([s_ref[0], s_ref[1]], axis=1)
        h = jnp.maximum(jnp.dot(sblk, w_ref[...],
                                preferred_element_type=jnp.float32)
                        + b_ref[...], 0.0)
        n = jnp.sqrt(jnp.sum(h * h, axis=1, keepdims=True))
        h = h / jnp.maximum(n, 1e-12)
        o_ref[0] = h[:, :HALF]
        o_ref[1] = h[:, HALF:]

    return pl.pallas_call(
        body,
        grid=(R // BR,),
        in_specs=[pl.BlockSpec((2, BR, HALF), lambda i: (0, i, 0)),
                  pl.BlockSpec((64, 64), lambda i: (0, 0)),
                  pl.BlockSpec((1, 64), lambda i: (0, 0))],
        out_specs=pl.BlockSpec((2, BR, HALF), lambda i: (0, i, 0)),
        out_shape=jax.ShapeDtypeStruct((2, R, HALF), jnp.float32),
    )(s_pair, W, b.reshape(1, 64))


def _tc_final(gb, gi, W3, b3):
    BM = 1024

    def body(gb_ref, gi_ref, w_ref, b_ref, o_ref):
        hb = jnp.dot(gb_ref[...], w_ref[...],
                     preferred_element_type=jnp.float32) + b_ref[...]
        hi = jnp.dot(gi_ref[...], w_ref[...],
                     preferred_element_type=jnp.float32) + b_ref[...]
        o_ref[...] = lax.dot_general(hb, hi, (((1,), (1,)), ((), ())),
                                     preferred_element_type=jnp.float32)

    return pl.pallas_call(
        body,
        grid=(4096 // BM,),
        in_specs=[pl.BlockSpec((1024, 64), lambda j: (0, 0)),
                  pl.BlockSpec((BM, 64), lambda j: (j, 0)),
                  pl.BlockSpec((64, 64), lambda j: (0, 0)),
                  pl.BlockSpec((1, 64), lambda j: (0, 0))],
        out_specs=pl.BlockSpec((1024, BM), lambda j: (0, j)),
        out_shape=jax.ShapeDtypeStruct((1024, 4096), jnp.float32),
    )(gb, gi, W3, b3.reshape(1, 64))


def _pad_edges(idx, vals, e_pad):
    e = vals.shape[0]
    pad = e_pad - e
    rows = jnp.concatenate([idx[0], jnp.zeros((pad,), idx.dtype)])
    cols = jnp.concatenate([idx[1], jnp.zeros((pad,), idx.dtype)])
    v = jnp.concatenate([vals, jnp.zeros((pad,), vals.dtype)])
    return (rows.reshape(-1, GCH).astype(jnp.int32),
            cols.reshape(-1, GCH).astype(jnp.int32), v)


def kernel(batch_idxes, A_indices, A_values, sa_indices, sa_values, item_idxes,
           item_emb, W1, b1, W2, b2, W3, b3):
    item_pair = jnp.stack([item_emb[:, :HALF], item_emb[:, HALF:]])
    sr, sc_, sv = _pad_edges(sa_indices, sa_values, _ES_PAD)
    sess = _spmm_sc(sr, sc_, sv, item_pair, _NSESS_PAD, _ES_PAD // NS // SB)

    x_pair = jnp.concatenate(
        [sess[:, :_NSESS], item_pair,
         jnp.zeros((2, _N_PAD - _N, HALF), jnp.float32)], axis=1)

    ar, ac, av = _pad_edges(A_indices, A_values, _EA_PAD)
    nblk = _EA_PAD // NS // SB
    s1 = _spmm_sc(ar, ac, av, x_pair, _N_PAD, nblk)
    h1 = _tc_mid(s1, W1, b1)
    s2 = _spmm_sc(ar, ac, av, h1, _N_PAD, nblk)
    h2 = _tc_mid(s2, W2, b2)
    s3 = _spmm_sc(ar, ac, av, h2, _N_PAD, nblk)

    gbp, gip = _sc_gather(s3, batch_idxes.astype(jnp.int32),
                          item_idxes.astype(jnp.int32))
    gb = jnp.concatenate([gbp[0], gbp[1]], axis=1)
    gi = jnp.concatenate([gip[0], gip[1]], axis=1)
    return _tc_final(gb, gi, W3, b3)


# speedup vs baseline: 1.0019x; 1.0019x over previous
import jax
import jax.numpy as jnp
from jax import lax
from jax.experimental import pallas as pl
from jax.experimental.pallas import tpu as pltpu
from jax.experimental.pallas import tpu_sc as plsc

NS = 16
EB = 512
SB = 2 * EB
GCH = 128
NGC = EB // GCH
HALF = 32
ZR = 200

_NSESS = 25000
_N = 50000
_NSESS_PAD = 25600
_N_PAD = 51200
_EA_PAD = 802816
_ES_PAD = 507904


def _spmm_sc(rows2d, cols2d, vals, x_pair, ndst_pad, nblk):
    ept = nblk * SB
    rpt = ndst_pad // NS
    ncp = rpt // ZR
    mesh = plsc.VectorSubcoreMesh(core_axis_name="c", subcore_axis_name="s")

    NQ = SB // GCH
    NSLOT = 4

    def body(rows_hbm, cols_hbm, vals_hbm, x_hbm, out_hbm,
             acc, gbuf, zbuf, rowv, colv, valv, gsem, ssem, zsem):
        c = lax.axis_index("c")
        s = lax.axis_index("s")

        def _zrow(r, carry):
            zbuf[r, 0:16] = jnp.zeros((16,), jnp.float32)
            zbuf[r, 16:32] = jnp.zeros((16,), jnp.float32)
            return carry
        lax.fori_loop(0, ZR, _zrow, 0)
        zdescs = [
            pltpu.async_copy(
                zbuf, acc.at[pl.ds(pl.multiple_of(s * rpt + k * ZR, ZR), ZR)],
                zsem)
            for k in range(ncp)
        ]
        for d in zdescs:
            d.wait()
        plsc.subcore_barrier()

        xh = x_hbm.at[c]
        dnums = lax.GatherDimensionNumbers(
            offset_dims=(), collapsed_slice_dims=(0,), start_index_map=(0,))

        def _sblk(b, carry):
            e0 = pl.multiple_of(s * ept + b * SB, SB)
            g0 = pl.multiple_of((s * ept) // GCH + b * NQ, NQ)
            pltpu.sync_copy(rows_hbm.at[pl.ds(g0, NQ)], rowv)
            pltpu.sync_copy(cols_hbm.at[pl.ds(g0, NQ)], colv)
            pltpu.sync_copy(vals_hbm.at[pl.ds(e0, SB)], valv)

            def _gather(q):
                return pltpu.async_copy(
                    xh.at[colv.at[q]],
                    gbuf.at[pl.ds((q % NSLOT) * GCH, GCH)], gsem)

            gd = {0: _gather(0)}
            sd = []
            for q in range(NQ):
                gd.pop(q).wait()
                if q + 1 < NQ:
                    if q + 1 >= NSLOT:
                        sd.pop(0).wait()
                    gd[q + 1] = _gather(q + 1)

                slot = (q % NSLOT) * GCH
                vbase = q * GCH

                @plsc.parallel_loop(0, GCH // 16, unroll=2)
                def _grp(g, _slot=slot, _vbase=vbase):
                    vv = valv[pl.ds(_vbase + g * 16, 16)]
                    base = _slot + g * 16
                    for k in range(16):
                        sp = lax.gather(
                            vv, jnp.full((16, 1), k, jnp.int32), dnums, (1,),
                            mode=lax.GatherScatterMode.PROMISE_IN_BOUNDS)
                        e = base + k
                        gbuf[e, 0:16] = gbuf[e, 0:16] * sp
                        gbuf[e, 16:32] = gbuf[e, 16:32] * sp

                sd.append(pltpu.async_copy(gbuf.at[pl.ds(slot, GCH)],
                                           acc.at[rowv.at[q]], ssem,
                                           add=True))
            for d in sd:
                d.wait()
            return carry
        lax.fori_loop(0, nblk, _sblk, 0)
        plsc.subcore_barrier()

        oh = out_hbm.at[c]
        for k in range(ncp):
            off = pl.multiple_of(s * rpt + k * ZR, ZR)
            pltpu.sync_copy(acc.at[pl.ds(off, ZR)], zbuf)
            pltpu.sync_copy(zbuf, oh.at[pl.ds(off, ZR)])

    f = pl.kernel(
        body,
        out_type=jax.ShapeDtypeStruct((2, ndst_pad, HALF), jnp.float32),
        mesh=mesh,
        scratch_types=[
            pltpu.VMEM_SHARED((ndst_pad, HALF), jnp.float32),
            pltpu.VMEM((EB, HALF), jnp.float32),
            pltpu.VMEM((ZR, HALF), jnp.float32),
            pltpu.VMEM((SB // GCH, GCH), jnp.int32),
            pltpu.VMEM((SB // GCH, GCH), jnp.int32),
            pltpu.VMEM((SB,), jnp.float32),
            pltpu.SemaphoreType.DMA,
            pltpu.SemaphoreType.DMA,
            pltpu.SemaphoreType.DMA,
        ],
        compiler_params=pltpu.CompilerParams(use_tc_tiling_on_sc=False),
    )
    return f(rows2d, cols2d, vals, x_pair)


def _sc_gather(s3, bidx, iidx):
    mesh = plsc.VectorSubcoreMesh(core_axis_name="c", subcore_axis_name="s")

    def body(s_hbm, b_hbm, i_hbm, gb_hbm, gi_hbm,
             bi_v, ii_a, ii_b, bbuf, ibuf, sem):
        c = lax.axis_index("c")
        s = lax.axis_index("s")
        src = s_hbm.at[c]
        o64 = pl.multiple_of(s * 64, 64)
        o256 = pl.multiple_of(s * 256, 256)
        pltpu.sync_copy(b_hbm.at[pl.ds(o64, 64)], bi_v)
        pltpu.async_copy(src.at[bi_v], bbuf, sem).wait()
        pltpu.sync_copy(bbuf, gb_hbm.at[c].at[pl.ds(o64, 64)])
        pltpu.sync_copy(i_hbm.at[pl.ds(o256, 128)], ii_a)
        pltpu.sync_copy(i_hbm.at[pl.ds(o256 + 128, 128)], ii_b)
        d0 = pltpu.async_copy(src.at[ii_a], ibuf.at[pl.ds(0, 128)], sem)
        d1 = pltpu.async_copy(src.at[ii_b], ibuf.at[pl.ds(128, 128)], sem)
        d0.wait()
        d1.wait()
        pltpu.sync_copy(ibuf, gi_hbm.at[c].at[pl.ds(o256, 256)])

    f = pl.kernel(
        body,
        out_type=(jax.ShapeDtypeStruct((2, 1024, HALF), jnp.float32),
                  jax.ShapeDtypeStruct((2, 4096, HALF), jnp.float32)),
        mesh=mesh,
        scratch_types=[
            pltpu.VMEM((64,), jnp.int32),
            pltpu.VMEM((128,), jnp.int32),
            pltpu.VMEM((128,), jnp.int32),
            pltpu.VMEM((64, HALF), jnp.float32),
            pltpu.VMEM((256, HALF), jnp.float32),
            pltpu.SemaphoreType.DMA,
        ],
        compiler_params=pltpu.CompilerParams(use_tc_tiling_on_sc=False),
    )
    return f(s3, bidx, iidx)


def _tc_mid(s_pair, W, b):
    R = s_pair.shape[1]
    BR = 2048

    def body(s_ref, w_ref, b_ref, o_ref):
        sblk = jnp.concatenate([s_ref[0], s_ref[1]], axis=1)
        h = jnp.maximum(jnp.dot(sblk, w_ref[...],
                                preferred_element_type=jnp.float32)
                        + b_ref[...], 0.0)
        n = jnp.sqrt(jnp.sum(h * h, axis=1, keepdims=True))
        h = h / jnp.maximum(n, 1e-12)
        o_ref[0] = h[:, :HALF]
        o_ref[1] = h[:, HALF:]

    return pl.pallas_call(
        body,
        grid=(R // BR,),
        in_specs=[pl.BlockSpec((2, BR, HALF), lambda i: (0, i, 0)),
                  pl.BlockSpec((64, 64), lambda i: (0, 0)),
                  pl.BlockSpec((1, 64), lambda i: (0, 0))],
        out_specs=pl.BlockSpec((2, BR, HALF), lambda i: (0, i, 0)),
        out_shape=jax.ShapeDtypeStruct((2, R, HALF), jnp.float32),
    )(s_pair, W, b.reshape(1, 64))


def _tc_final(gb, gi, W3, b3):
    BM = 1024

    def body(gb_ref, gi_ref, w_ref, b_ref, o_ref):
        hb = jnp.dot(gb_ref[...], w_ref[...],
                     preferred_element_type=jnp.float32) + b_ref[...]
        hi = jnp.dot(gi_ref[...], w_ref[...],
                     preferred_element_type=jnp.float32) + b_ref[...]
        o_ref[...] = lax.dot_general(hb, hi, (((1,), (1,)), ((), ())),
                                     preferred_element_type=jnp.float32)

    return pl.pallas_call(
        body,
        grid=(4096 // BM,),
        in_specs=[pl.BlockSpec((1024, 64), lambda j: (0, 0)),
                  pl.BlockSpec((BM, 64), lambda j: (j, 0)),
                  pl.BlockSpec((64, 64), lambda j: (0, 0)),
                  pl.BlockSpec((1, 64), lambda j: (0, 0))],
        out_specs=pl.BlockSpec((1024, BM), lambda j: (0, j)),
        out_shape=jax.ShapeDtypeStruct((1024, 4096), jnp.float32),
    )(gb, gi, W3, b3.reshape(1, 64))


def _pad_edges(idx, vals, e_pad):
    e = vals.shape[0]
    pad = e_pad - e
    rows = jnp.concatenate([idx[0], jnp.zeros((pad,), idx.dtype)])
    cols = jnp.concatenate([idx[1], jnp.zeros((pad,), idx.dtype)])
    v = jnp.concatenate([vals, jnp.zeros((pad,), vals.dtype)])
    return (rows.reshape(-1, GCH).astype(jnp.int32),
            cols.reshape(-1, GCH).astype(jnp.int32), v)


def kernel(batch_idxes, A_indices, A_values, sa_indices, sa_values, item_idxes,
           item_emb, W1, b1, W2, b2, W3, b3):
    item_pair = jnp.stack([item_emb[:, :HALF], item_emb[:, HALF:]])
    sr, sc_, sv = _pad_edges(sa_indices, sa_values, _ES_PAD)
    sess = _spmm_sc(sr, sc_, sv, item_pair, _NSESS_PAD, _ES_PAD // NS // SB)

    x_pair = jnp.concatenate(
        [sess[:, :_NSESS], item_pair,
         jnp.zeros((2, _N_PAD - _N, HALF), jnp.float32)], axis=1)

    ar, ac, av = _pad_edges(A_indices, A_values, _EA_PAD)
    nblk = _EA_PAD // NS // SB
    s1 = _spmm_sc(ar, ac, av, x_pair, _N_PAD, nblk)
    h1 = _tc_mid(s1, W1, b1)
    s2 = _spmm_sc(ar, ac, av, h1, _N_PAD, nblk)
    h2 = _tc_mid(s2, W2, b2)
    s3 = _spmm_sc(ar, ac, av, h2, _N_PAD, nblk)

    gbp, gip = _sc_gather(s3, batch_idxes.astype(jnp.int32),
                          item_idxes.astype(jnp.int32))
    gb = jnp.concatenate([gbp[0], gbp[1]], axis=1)
    gi = jnp.concatenate([gip[0], gip[1]], axis=1)
    return _tc_final(gb, gi, W3, b3)


# speedup vs baseline: 1.0030x; 1.0011x over previous
import jax
import jax.numpy as jnp
from jax import lax
from jax.experimental import pallas as pl
from jax.experimental.pallas import tpu as pltpu
from jax.experimental.pallas import tpu_sc as plsc

NS = 16
EB = 512
SB = 2 * EB
GCH = 128
NGC = EB // GCH
HALF = 32
ZR = 200

_NSESS = 25000
_N = 50000
_NSESS_PAD = 25600
_N_PAD = 51200
_EA_PAD = 802816
_ES_PAD = 507904


def _spmm_sc(rows2d, cols2d, vals, x_pair, ndst_pad, nblk):
    ept = nblk * SB
    rpt = ndst_pad // NS
    ncp = rpt // ZR
    mesh = plsc.VectorSubcoreMesh(core_axis_name="c", subcore_axis_name="s")

    NQ = SB // GCH
    NSLOT = 4

    def body(rows_hbm, cols_hbm, vals_hbm, x_hbm, out_hbm,
             acc, gbuf, zbuf, rowv, colv, valv, gsem, ssem, zsem):
        c = lax.axis_index("c")
        s = lax.axis_index("s")

        def _zrow(r, carry):
            zbuf[r, 0:16] = jnp.zeros((16,), jnp.float32)
            zbuf[r, 16:32] = jnp.zeros((16,), jnp.float32)
            return carry
        lax.fori_loop(0, ZR, _zrow, 0)
        zdescs = [
            pltpu.async_copy(
                zbuf, acc.at[pl.ds(pl.multiple_of(s * rpt + k * ZR, ZR), ZR)],
                zsem)
            for k in range(ncp)
        ]
        for d in zdescs:
            d.wait()
        plsc.subcore_barrier()

        xh = x_hbm.at[c]
        dnums = lax.GatherDimensionNumbers(
            offset_dims=(), collapsed_slice_dims=(0,), start_index_map=(0,))

        def _sblk(b, carry):
            e0 = pl.multiple_of(s * ept + b * SB, SB)
            g0 = pl.multiple_of((s * ept) // GCH + b * NQ, NQ)
            pltpu.sync_copy(rows_hbm.at[pl.ds(g0, NQ)], rowv)
            pltpu.sync_copy(cols_hbm.at[pl.ds(g0, NQ)], colv)
            pltpu.sync_copy(vals_hbm.at[pl.ds(e0, SB)], valv)

            def _gather(q):
                return pltpu.async_copy(
                    xh.at[colv.at[q]],
                    gbuf.at[pl.ds((q % NSLOT) * GCH, GCH)], gsem)

            gd = {0: _gather(0)}
            sd = []
            for q in range(NQ):
                gd.pop(q).wait()
                if q + 1 < NQ:
                    if q + 1 >= NSLOT:
                        sd.pop(0).wait()
                    gd[q + 1] = _gather(q + 1)

                slot = (q % NSLOT) * GCH
                vbase = q * GCH

                @plsc.parallel_loop(0, GCH // 16, unroll=2)
                def _grp(g, _slot=slot, _vbase=vbase):
                    base = _slot + g * 16
                    vv = valv[pl.ds(_vbase + g * 16, 16)]
                    for k in range(16):
                        sp = vv[k]
                        e = base + k
                        gbuf[e, 0:16] = gbuf[e, 0:16] * sp
                        gbuf[e, 16:32] = gbuf[e, 16:32] * sp

                sd.append(pltpu.async_copy(gbuf.at[pl.ds(slot, GCH)],
                                           acc.at[rowv.at[q]], ssem,
                                           add=True))
            for d in sd:
                d.wait()
            return carry
        lax.fori_loop(0, nblk, _sblk, 0)
        plsc.subcore_barrier()

        oh = out_hbm.at[c]
        for k in range(ncp):
            off = pl.multiple_of(s * rpt + k * ZR, ZR)
            pltpu.sync_copy(acc.at[pl.ds(off, ZR)], zbuf)
            pltpu.sync_copy(zbuf, oh.at[pl.ds(off, ZR)])

    f = pl.kernel(
        body,
        out_type=jax.ShapeDtypeStruct((2, ndst_pad, HALF), jnp.float32),
        mesh=mesh,
        scratch_types=[
            pltpu.VMEM_SHARED((ndst_pad, HALF), jnp.float32),
            pltpu.VMEM((EB, HALF), jnp.float32),
            pltpu.VMEM((ZR, HALF), jnp.float32),
            pltpu.VMEM((SB // GCH, GCH), jnp.int32),
            pltpu.VMEM((SB // GCH, GCH), jnp.int32),
            pltpu.VMEM((SB,), jnp.float32),
            pltpu.SemaphoreType.DMA,
            pltpu.SemaphoreType.DMA,
            pltpu.SemaphoreType.DMA,
        ],
        compiler_params=pltpu.CompilerParams(use_tc_tiling_on_sc=False),
    )
    return f(rows2d, cols2d, vals, x_pair)


def _sc_gather(s3, bidx, iidx):
    mesh = plsc.VectorSubcoreMesh(core_axis_name="c", subcore_axis_name="s")

    def body(s_hbm, b_hbm, i_hbm, gb_hbm, gi_hbm,
             bi_v, ii_a, ii_b, bbuf, ibuf, sem):
        c = lax.axis_index("c")
        s = lax.axis_index("s")
        src = s_hbm.at[c]
        o64 = pl.multiple_of(s * 64, 64)
        o256 = pl.multiple_of(s * 256, 256)
        pltpu.sync_copy(b_hbm.at[pl.ds(o64, 64)], bi_v)
        pltpu.async_copy(src.at[bi_v], bbuf, sem).wait()
        pltpu.sync_copy(bbuf, gb_hbm.at[c].at[pl.ds(o64, 64)])
        pltpu.sync_copy(i_hbm.at[pl.ds(o256, 128)], ii_a)
        pltpu.sync_copy(i_hbm.at[pl.ds(o256 + 128, 128)], ii_b)
        d0 = pltpu.async_copy(src.at[ii_a], ibuf.at[pl.ds(0, 128)], sem)
        d1 = pltpu.async_copy(src.at[ii_b], ibuf.at[pl.ds(128, 128)], sem)
        d0.wait()
        d1.wait()
        pltpu.sync_copy(ibuf, gi_hbm.at[c].at[pl.ds(o256, 256)])

    f = pl.kernel(
        body,
        out_type=(jax.ShapeDtypeStruct((2, 1024, HALF), jnp.float32),
                  jax.ShapeDtypeStruct((2, 4096, HALF), jnp.float32)),
        mesh=mesh,
        scratch_types=[
            pltpu.VMEM((64,), jnp.int32),
            pltpu.VMEM((128,), jnp.int32),
            pltpu.VMEM((128,), jnp.int32),
            pltpu.VMEM((64, HALF), jnp.float32),
            pltpu.VMEM((256, HALF), jnp.float32),
            pltpu.SemaphoreType.DMA,
        ],
        compiler_params=pltpu.CompilerParams(use_tc_tiling_on_sc=False),
    )
    return f(s3, bidx, iidx)


def _tc_mid(s_pair, W, b):
    R = s_pair.shape[1]
    BR = 2048

    def body(s_ref, w_ref, b_ref, o_ref):
        sblk = jnp.concatenate([s_ref[0], s_ref[1]], axis=1)
        h = jnp.maximum(jnp.dot(sblk, w_ref[...],
                                preferred_element_type=jnp.float32)
                        + b_ref[...], 0.0)
        n = jnp.sqrt(jnp.sum(h * h, axis=1, keepdims=True))
        h = h / jnp.maximum(n, 1e-12)
        o_ref[0] = h[:, :HALF]
        o_ref[1] = h[:, HALF:]

    return pl.pallas_call(
        body,
        grid=(R // BR,),
        in_specs=[pl.BlockSpec((2, BR, HALF), lambda i: (0, i, 0)),
                  pl.BlockSpec((64, 64), lambda i: (0, 0)),
                  pl.BlockSpec((1, 64), lambda i: (0, 0))],
        out_specs=pl.BlockSpec((2, BR, HALF), lambda i: (0, i, 0)),
        out_shape=jax.ShapeDtypeStruct((2, R, HALF), jnp.float32),
    )(s_pair, W, b.reshape(1, 64))


def _tc_final(gb, gi, W3, b3):
    BM = 1024

    def body(gb_ref, gi_ref, w_ref, b_ref, o_ref):
        hb = jnp.dot(gb_ref[...], w_ref[...],
                     preferred_element_type=jnp.float32) + b_ref[...]
        hi = jnp.dot(gi_ref[...], w_ref[...],
                     preferred_element_type=jnp.float32) + b_ref[...]
        o_ref[...] = lax.dot_general(hb, hi, (((1,), (1,)), ((), ())),
                                     preferred_element_type=jnp.float32)

    return pl.pallas_call(
        body,
        grid=(4096 // BM,),
        in_specs=[pl.BlockSpec((1024, 64), lambda j: (0, 0)),
                  pl.BlockSpec((BM, 64), lambda j: (j, 0)),
                  pl.BlockSpec((64, 64), lambda j: (0, 0)),
                  pl.BlockSpec((1, 64), lambda j: (0, 0))],
        out_specs=pl.BlockSpec((1024, BM), lambda j: (0, j)),
        out_shape=jax.ShapeDtypeStruct((1024, 4096), jnp.float32),
    )(gb, gi, W3, b3.reshape(1, 64))


def _pad_edges(idx, vals, e_pad):
    e = vals.shape[0]
    pad = e_pad - e
    rows = jnp.concatenate([idx[0], jnp.zeros((pad,), idx.dtype)])
    cols = jnp.concatenate([idx[1], jnp.zeros((pad,), idx.dtype)])
    v = jnp.concatenate([vals, jnp.zeros((pad,), vals.dtype)])
    return (rows.reshape(-1, GCH).astype(jnp.int32),
            cols.reshape(-1, GCH).astype(jnp.int32), v)


def kernel(batch_idxes, A_indices, A_values, sa_indices, sa_values, item_idxes,
           item_emb, W1, b1, W2, b2, W3, b3):
    item_pair = jnp.stack([item_emb[:, :HALF], item_emb[:, HALF:]])
    sr, sc_, sv = _pad_edges(sa_indices, sa_values, _ES_PAD)
    sess = _spmm_sc(sr, sc_, sv, item_pair, _NSESS_PAD, _ES_PAD // NS // SB)

    x_pair = jnp.concatenate(
        [sess[:, :_NSESS], item_pair,
         jnp.zeros((2, _N_PAD - _N, HALF), jnp.float32)], axis=1)

    ar, ac, av = _pad_edges(A_indices, A_values, _EA_PAD)
    nblk = _EA_PAD // NS // SB
    s1 = _spmm_sc(ar, ac, av, x_pair, _N_PAD, nblk)
    h1 = _tc_mid(s1, W1, b1)
    s2 = _spmm_sc(ar, ac, av, h1, _N_PAD, nblk)
    h2 = _tc_mid(s2, W2, b2)
    s3 = _spmm_sc(ar, ac, av, h2, _N_PAD, nblk)

    gbp, gip = _sc_gather(s3, batch_idxes.astype(jnp.int32),
                          item_idxes.astype(jnp.int32))
    gb = jnp.concatenate([gbp[0], gbp[1]], axis=1)
    gi = jnp.concatenate([gip[0], gip[1]], axis=1)
    return _tc_final(gb, gi, W3, b3)


# speedup vs baseline: 1.1004x; 1.0971x over previous
import jax
import jax.numpy as jnp
from jax import lax
from jax.experimental import pallas as pl
from jax.experimental.pallas import tpu as pltpu
from jax.experimental.pallas import tpu_sc as plsc

NS = 16
EB = 512
SB = 2 * EB
GCH = 128
NGC = EB // GCH
NQ = SB // GCH
HALF = 32
ZR = 200

_NSESS = 25000
_N = 50000
_NSESS_PAD = 25600
_N_PAD = 51200
_EA_PAD = 802816
_ES_PAD = 507904


def _spmm_sc(rows2d, cols2d, vals, x_pair, ndst_pad, nblk, gather_idx=None):
    ept = nblk * SB
    rpt = ndst_pad // NS
    ncp = rpt // ZR
    final = gather_idx is not None
    mesh = plsc.VectorSubcoreMesh(core_axis_name="c", subcore_axis_name="s")

    def body(*refs):
        if final:
            (rows_hbm, cols_hbm, vals_hbm, x_hbm, b_hbm, i_hbm,
             gb_hbm, gi_hbm,
             acc, gbuf, zbuf, rowv, colv, valv, gsem, zsem) = refs
        else:
            (rows_hbm, cols_hbm, vals_hbm, x_hbm, out_hbm,
             acc, gbuf, zbuf, rowv, colv, valv, gsem, zsem) = refs
        c = lax.axis_index("c")
        s = lax.axis_index("s")

        def _zrow(r, carry):
            zbuf[r, 0:16] = jnp.zeros((16,), jnp.float32)
            zbuf[r, 16:32] = jnp.zeros((16,), jnp.float32)
            return carry
        lax.fori_loop(0, ZR, _zrow, 0)
        zdescs = [
            pltpu.async_copy(
                zbuf, acc.at[pl.ds(pl.multiple_of(s * rpt + k * ZR, ZR), ZR)],
                zsem)
            for k in range(ncp)
        ]
        for d in zdescs:
            d.wait()
        plsc.subcore_barrier()

        xh = x_hbm.at[c]

        def _sblk(b, carry):
            e0 = pl.multiple_of(s * ept + b * SB, SB)
            g0 = pl.multiple_of((s * ept) // GCH + b * NQ, NQ)
            pltpu.sync_copy(rows_hbm.at[pl.ds(g0, NQ)], rowv)
            pltpu.sync_copy(cols_hbm.at[pl.ds(g0, NQ)], colv)
            pltpu.sync_copy(vals_hbm.at[pl.ds(e0, SB)], valv)
            for sub in range(2):
                descs = [
                    pltpu.async_copy(xh.at[colv.at[sub * NGC + j]],
                                     gbuf.at[pl.ds(j * GCH, GCH)], gsem)
                    for j in range(NGC)
                ]
                for d in descs:
                    d.wait()

                vbase = sub * EB

                @plsc.parallel_loop(0, EB // 16, unroll=2)
                def _grp(g, _vbase=vbase):
                    base = g * 16
                    vv = valv[pl.ds(_vbase + g * 16, 16)]
                    for k in range(16):
                        sp = vv[k]
                        e = base + k
                        gbuf[e, 0:16] = gbuf[e, 0:16] * sp
                        gbuf[e, 16:32] = gbuf[e, 16:32] * sp

                for j in range(NGC):
                    pltpu.sync_copy(gbuf.at[pl.ds(j * GCH, GCH)],
                                    acc.at[rowv.at[sub * NGC + j]], add=True)
            return carry
        lax.fori_loop(0, nblk, _sblk, 0)
        plsc.subcore_barrier()

        if final:
            o64 = pl.multiple_of(s * 64, 64)
            o256 = pl.multiple_of(s * 256, 256)
            bi_v = rowv.at[0].at[pl.ds(0, 64)]
            ii_a = colv.at[0]
            ii_b = colv.at[1]
            bbuf = gbuf.at[pl.ds(0, 64)]
            ibuf = gbuf.at[pl.ds(64, 256)]
            pltpu.sync_copy(b_hbm.at[pl.ds(o64, 64)], bi_v)
            pltpu.sync_copy(i_hbm.at[pl.ds(o256, 128)], ii_a)
            pltpu.sync_copy(i_hbm.at[pl.ds(o256 + 128, 128)], ii_b)
            d0 = pltpu.async_copy(acc.at[bi_v], bbuf, gsem)
            d1 = pltpu.async_copy(acc.at[ii_a], ibuf.at[pl.ds(0, 128)], gsem)
            d2 = pltpu.async_copy(acc.at[ii_b], ibuf.at[pl.ds(128, 128)], gsem)
            d0.wait()
            d1.wait()
            d2.wait()
            pltpu.sync_copy(bbuf, gb_hbm.at[c].at[pl.ds(o64, 64)])
            pltpu.sync_copy(ibuf, gi_hbm.at[c].at[pl.ds(o256, 256)])
        else:
            oh = out_hbm.at[c]
            for k in range(ncp):
                off = pl.multiple_of(s * rpt + k * ZR, ZR)
                pltpu.sync_copy(acc.at[pl.ds(off, ZR)], zbuf)
                pltpu.sync_copy(zbuf, oh.at[pl.ds(off, ZR)])

    if final:
        out_type = (jax.ShapeDtypeStruct((2, 1024, HALF), jnp.float32),
                    jax.ShapeDtypeStruct((2, 4096, HALF), jnp.float32))
    else:
        out_type = jax.ShapeDtypeStruct((2, ndst_pad, HALF), jnp.float32)

    f = pl.kernel(
        body,
        out_type=out_type,
        mesh=mesh,
        scratch_types=[
            pltpu.VMEM_SHARED((ndst_pad, HALF), jnp.float32),
            pltpu.VMEM((EB, HALF), jnp.float32),
            pltpu.VMEM((ZR, HALF), jnp.float32),
            pltpu.VMEM((NQ, GCH), jnp.int32),
            pltpu.VMEM((NQ, GCH), jnp.int32),
            pltpu.VMEM((SB,), jnp.float32),
            pltpu.SemaphoreType.DMA,
            pltpu.SemaphoreType.DMA,
        ],
        compiler_params=pltpu.CompilerParams(use_tc_tiling_on_sc=False),
    )
    if final:
        return f(rows2d, cols2d, vals, x_pair, gather_idx[0], gather_idx[1])
    return f(rows2d, cols2d, vals, x_pair)


def _tc_mid(s_pair, W, b):
    R = s_pair.shape[1]
    BR = 2048

    def body(s_ref, w_ref, b_ref, o_ref):
        sblk = jnp.concatenate([s_ref[0], s_ref[1]], axis=1)
        h = jnp.maximum(jnp.dot(sblk, w_ref[...],
                                preferred_element_type=jnp.float32)
                        + b_ref[...], 0.0)
        n = jnp.sqrt(jnp.sum(h * h, axis=1, keepdims=True))
        h = h / jnp.maximum(n, 1e-12)
        o_ref[0] = h[:, :HALF]
        o_ref[1] = h[:, HALF:]

    return pl.pallas_call(
        body,
        grid=(R // BR,),
        in_specs=[pl.BlockSpec((2, BR, HALF), lambda i: (0, i, 0)),
                  pl.BlockSpec((64, 64), lambda i: (0, 0)),
                  pl.BlockSpec((1, 64), lambda i: (0, 0))],
        out_specs=pl.BlockSpec((2, BR, HALF), lambda i: (0, i, 0)),
        out_shape=jax.ShapeDtypeStruct((2, R, HALF), jnp.float32),
    )(s_pair, W, b.reshape(1, 64))


def _tc_final(gbp, gip, W3, b3):
    BM = 1024

    def body(gb_ref, gi_ref, w_ref, b_ref, o_ref):
        gb = jnp.concatenate([gb_ref[0], gb_ref[1]], axis=1)
        gi = jnp.concatenate([gi_ref[0], gi_ref[1]], axis=1)
        hb = jnp.dot(gb, w_ref[...],
                     preferred_element_type=jnp.float32) + b_ref[...]
        hi = jnp.dot(gi, w_ref[...],
                     preferred_element_type=jnp.float32) + b_ref[...]
        o_ref[...] = lax.dot_general(hb, hi, (((1,), (1,)), ((), ())),
                                     preferred_element_type=jnp.float32)

    return pl.pallas_call(
        body,
        grid=(4096 // BM,),
        in_specs=[pl.BlockSpec((2, 1024, HALF), lambda j: (0, 0, 0)),
                  pl.BlockSpec((2, BM, HALF), lambda j: (0, j, 0)),
                  pl.BlockSpec((64, 64), lambda j: (0, 0)),
                  pl.BlockSpec((1, 64), lambda j: (0, 0))],
        out_specs=pl.BlockSpec((1024, BM), lambda j: (0, j)),
        out_shape=jax.ShapeDtypeStruct((1024, 4096), jnp.float32),
    )(gbp, gip, W3, b3.reshape(1, 64))


def _pad_edges(idx, vals, e_pad):
    e = vals.shape[0]
    pad = e_pad - e
    rows = jnp.concatenate([idx[0], jnp.zeros((pad,), idx.dtype)])
    cols = jnp.concatenate([idx[1], jnp.zeros((pad,), idx.dtype)])
    v = jnp.concatenate([vals, jnp.zeros((pad,), vals.dtype)])
    return (rows.reshape(-1, GCH).astype(jnp.int32),
            cols.reshape(-1, GCH).astype(jnp.int32), v)


def kernel(batch_idxes, A_indices, A_values, sa_indices, sa_values, item_idxes,
           item_emb, W1, b1, W2, b2, W3, b3):
    item_pair = jnp.stack([item_emb[:, :HALF], item_emb[:, HALF:]])
    sr, sc_, sv = _pad_edges(sa_indices, sa_values, _ES_PAD)
    sess = _spmm_sc(sr, sc_, sv, item_pair, _NSESS_PAD, _ES_PAD // NS // SB)

    x_pair = jnp.concatenate(
        [sess[:, :_NSESS], item_pair,
         jnp.zeros((2, _N_PAD - _N, HALF), jnp.float32)], axis=1)

    ar, ac, av = _pad_edges(A_indices, A_values, _EA_PAD)
    nblk = _EA_PAD // NS // SB
    s1 = _spmm_sc(ar, ac, av, x_pair, _N_PAD, nblk)
    h1 = _tc_mid(s1, W1, b1)
    s2 = _spmm_sc(ar, ac, av, h1, _N_PAD, nblk)
    h2 = _tc_mid(s2, W2, b2)
    gbp, gip = _spmm_sc(ar, ac, av, h2, _N_PAD, nblk,
                        gather_idx=(batch_idxes.astype(jnp.int32),
                                    item_idxes.astype(jnp.int32)))
    return _tc_final(gbp, gip, W3, b3)
